# Initial kernel scaffold; baseline (speedup 1.0000x reference)
import functools
from functools import partial

import jax
import jax.numpy as jnp
from jax import lax
from jax.experimental import pallas as pl
from jax.experimental.pallas import tpu as pltpu
from jax.experimental.pallas import tpu_sc as plsc

NW = 16
K = 128


def _cdiv(a, b):
    return (a + b - 1) // b


def _make_prop(n, c, cpw, niters, lo, hi):
    assert n % NW == 0
    npw = n // NW
    cb = next(d for d in range(min(npw, 128), 0, -1) if npw % d == 0)
    ncb = npw // cb
    zpw = _cdiv(n + 1, NW * 128)
    n_acc = NW * 128 * zpw
    mesh = plsc.VectorSubcoreMesh(core_axis_name="c", subcore_axis_name="s",
                                  num_cores=1)

    @partial(
        pl.kernel,
        out_type=(jax.ShapeDtypeStruct((n, c), jnp.float32),
                  jax.ShapeDtypeStruct((n, c), jnp.float32)),
        mesh=mesh,
        scratch_types=[
            pltpu.VMEM((cpw, K), jnp.int32),
            pltpu.VMEM((cpw, K), jnp.int32),
            pltpu.VMEM((K, c), jnp.float32),
            pltpu.VMEM((K, c), jnp.float32),
            pltpu.VMEM((128, c), jnp.float32),
            pltpu.VMEM((cb, c), jnp.float32),
            pltpu.VMEM((cb, c), jnp.float32),
            pltpu.VMEM((cb, c), jnp.float32),
            pltpu.VMEM((cb, c), jnp.float32),
            pltpu.VMEM_SHARED((n_acc, c), jnp.float32),
            pltpu.SemaphoreType.DMA,
            pltpu.SemaphoreType.DMA,
        ],
    )
    def prop(g0, talpha, res, disb, src3, dst3, h_out, g_out,
             sidx, didx, bufa, bufb, zbuf, accb, tb, rb, db, acc,
             sema, semb):
        w = lax.axis_index("s")

        pltpu.sync_copy(src3.at[w], sidx)
        pltpu.sync_copy(dst3.at[w], didx)

        def zinit(r, carry):
            for c4 in range(c // 16):
                zbuf[r, pl.ds(c4 * 16, 16)] = jnp.zeros((16,), jnp.float32)
            return carry
        lax.fori_loop(0, 128, zinit, 0)

        for j in range(ncb):
            r0 = w * npw + j * cb
            pltpu.sync_copy(g0.at[pl.ds(r0, cb)], accb)
            pltpu.sync_copy(accb, g_out.at[pl.ds(r0, cb)])
        plsc.subcore_barrier()

        def one_iter(i, carry):
            for z in range(zpw):
                pltpu.sync_copy(zbuf,
                                acc.at[pl.ds(w * 128 * zpw + z * 128, 128)])
            plsc.subcore_barrier()

            def pair(p, carry2):
                c0 = 2 * p
                da = pltpu.async_copy(g_out.at[sidx.at[c0]], bufa, sema)
                dbc = pltpu.async_copy(g_out.at[sidx.at[c0 + 1]], bufb, semb)
                da.wait()
                pltpu.sync_copy(bufa, acc.at[didx.at[c0]], add=True)
                dbc.wait()
                pltpu.sync_copy(bufb, acc.at[didx.at[c0 + 1]], add=True)
                return carry2
            lax.fori_loop(0, cpw // 2, pair, 0)
            plsc.subcore_barrier()

            for j in range(ncb):
                r0 = w * npw + j * cb
                pltpu.sync_copy(acc.at[pl.ds(r0, cb)], accb)
                pltpu.sync_copy(talpha.at[pl.ds(r0, cb)], tb)
                pltpu.sync_copy(res.at[pl.ds(r0, cb)], rb)
                pltpu.sync_copy(disb.at[pl.ds(r0, cb)], db)

                def crow(r, carry3):
                    for c4 in range(c // 16):
                        sl = pl.ds(c4 * 16, 16)
                        h = jnp.minimum(
                            jnp.maximum(tb[r, sl] * accb[r, sl] + rb[r, sl],
                                        lo), hi)
                        accb[r, sl] = h
                        tb[r, sl] = db[r, sl] * h
                    return carry3
                lax.fori_loop(0, cb, crow, 0)

                pltpu.sync_copy(tb, g_out.at[pl.ds(r0, cb)])

                @pl.when(i == niters - 1)
                def _():
                    pltpu.sync_copy(accb, h_out.at[pl.ds(r0, cb)])
            plsc.subcore_barrier()
            return carry
        lax.fori_loop(0, niters, one_iter, 0)

    return prop


def _make_deg(n, cpw):
    assert n % NW == 0
    npw = n // NW
    cb = next(d for d in range(min(npw, 128), 0, -1) if npw % d == 0)
    ncb = npw // cb
    zpw = _cdiv(n + 1, NW * 128)
    n_acc = NW * 128 * zpw
    mesh = plsc.VectorSubcoreMesh(core_axis_name="c", subcore_axis_name="s",
                                  num_cores=1)

    @partial(
        pl.kernel,
        out_type=jax.ShapeDtypeStruct((n, 16), jnp.float32),
        mesh=mesh,
        scratch_types=[
            pltpu.VMEM((cpw, K), jnp.int32),
            pltpu.VMEM((K, 16), jnp.float32),
            pltpu.VMEM((128, 16), jnp.float32),
            pltpu.VMEM((cb, 16), jnp.float32),
            pltpu.VMEM_SHARED((n_acc, 16), jnp.float32),
        ],
    )
    def deg(dst3, out, didx, ones, zbuf, ob, acc):
        w = lax.axis_index("s")
        pltpu.sync_copy(dst3.at[w], didx)

        def finit(r, carry):
            ones[r, pl.ds(0, 16)] = jnp.ones((16,), jnp.float32)
            return carry
        lax.fori_loop(0, K, finit, 0)

        def zinit(r, carry):
            zbuf[r, pl.ds(0, 16)] = jnp.zeros((16,), jnp.float32)
            return carry
        lax.fori_loop(0, 128, zinit, 0)

        for z in range(zpw):
            pltpu.sync_copy(zbuf, acc.at[pl.ds(w * 128 * zpw + z * 128, 128)])
        plsc.subcore_barrier()

        def scat(cc, carry):
            pltpu.sync_copy(ones, acc.at[didx.at[cc]], add=True)
            return carry
        lax.fori_loop(0, cpw, scat, 0)
        plsc.subcore_barrier()

        for j in range(ncb):
            r0 = w * npw + j * cb
            pltpu.sync_copy(acc.at[pl.ds(r0, cb)], ob)
            pltpu.sync_copy(ob, out.at[pl.ds(r0, cb)])

    return deg


def _tc_matmul(x, wmat):
    m, d = x.shape
    cc = wmat.shape[1]
    bm = 500

    def body(xr, wr, orf):
        orf[...] = jnp.dot(xr[...], wr[...],
                           preferred_element_type=jnp.float32)

    return pl.pallas_call(
        body,
        grid=(m // bm,),
        in_specs=[pl.BlockSpec((bm, d), lambda i: (i, 0)),
                  pl.BlockSpec((d, cc), lambda i: (0, 0))],
        out_specs=pl.BlockSpec((bm, cc), lambda i: (i, 0)),
        out_shape=jax.ShapeDtypeStruct((m, cc), jnp.float32),
    )(x, wmat)


def _tc_prep(logits, mask_b, lab_b):
    m, cc = logits.shape
    bm = 500

    def body(lr, mr, br, pr, er):
        z = lr[...]
        zm = jnp.max(z, axis=1, keepdims=True)
        ez = jnp.exp(z - zm)
        p = ez / jnp.sum(ez, axis=1, keepdims=True)
        oh = (br[...] == lax.broadcasted_iota(jnp.int32, (bm, cc), 1)
              ).astype(jnp.float32)
        pr[...] = p
        er[...] = jnp.where(mr[...] != 0, oh - p, 0.0)

    return pl.pallas_call(
        body,
        grid=(m // bm,),
        in_specs=[pl.BlockSpec((bm, cc), lambda i: (i, 0))] * 3,
        out_specs=[pl.BlockSpec((bm, cc), lambda i: (i, 0))] * 2,
        out_shape=[jax.ShapeDtypeStruct((m, cc), jnp.float32)] * 2,
    )(logits, mask_b, lab_b)


def _tc_mid(probs, smerr, mask_b, lab_b):
    m, cc = probs.shape
    bm = 500

    def body(pr, sr, mr, br, yr):
        oh = (br[...] == lax.broadcasted_iota(jnp.int32, (bm, cc), 1)
              ).astype(jnp.float32)
        yr[...] = jnp.where(mr[...] != 0, oh, pr[...] + sr[...])

    return pl.pallas_call(
        body,
        grid=(m // bm,),
        in_specs=[pl.BlockSpec((bm, cc), lambda i: (i, 0))] * 4,
        out_specs=pl.BlockSpec((bm, cc), lambda i: (i, 0)),
        out_shape=jax.ShapeDtypeStruct((m, cc), jnp.float32),
    )(probs, smerr, mask_b, lab_b)


def _tc_logclip(h):
    m, cc = h.shape
    bm = 500

    def body(hr, orf):
        orf[...] = jnp.log(jnp.maximum(hr[...], 1e-15))

    return pl.pallas_call(
        body,
        grid=(m // bm,),
        in_specs=[pl.BlockSpec((bm, cc), lambda i: (i, 0))],
        out_specs=pl.BlockSpec((bm, cc), lambda i: (i, 0)),
        out_shape=jax.ShapeDtypeStruct((m, cc), jnp.float32),
    )(h)


NUM_CORRECTION_LAYERS = 50
CORRECTION_ALPHA = 0.5
NUM_SMOOTHING_LAYERS = 50
SMOOTHING_ALPHA = 0.8
SCALE = 1.0


def kernel(x, edge_index, W, train_mask, train_labels):
    n, d = x.shape
    c = W.shape[1]
    e = edge_index.shape[1]
    src = edge_index[0].astype(jnp.int32)
    dst = edge_index[1].astype(jnp.int32)

    cpw = _cdiv(e, NW * K)
    cpw += cpw % 2
    e_pad = NW * cpw * K
    src_p = jnp.concatenate([src, jnp.zeros((e_pad - e,), jnp.int32)])
    dst_p = jnp.concatenate([dst, jnp.full((e_pad - e,), n, jnp.int32)])
    src3 = src_p.reshape(NW, cpw, K)
    dst3 = dst_p.reshape(NW, cpw, K)

    deg = _make_deg(n, cpw)(dst3)[:, 0]
    dis = jnp.where(deg > 0, lax.rsqrt(jnp.maximum(deg, 1e-12)), 0.0)
    disb = jnp.broadcast_to(dis[:, None], (n, c))

    xw = _tc_matmul(x, W)
    zeros = jnp.zeros((n, c), jnp.float32)
    big = jnp.float32(3e38)

    logits, _ = _make_prop(n, c, cpw, 1, -big, big)(
        disb * xw, disb, zeros, disb, src3, dst3)

    mask_b = jnp.broadcast_to(
        train_mask.astype(jnp.int32)[:, None], (n, c))
    lab_b = jnp.broadcast_to(
        train_labels.astype(jnp.int32)[:, None], (n, c))
    probs, err = _tc_prep(logits, mask_b, lab_b)

    smerr, _ = _make_prop(n, c, cpw, NUM_CORRECTION_LAYERS, -1.0, 1.0)(
        disb * err, CORRECTION_ALPHA * disb,
        (1.0 - CORRECTION_ALPHA) * err, disb, src3, dst3)

    y = _tc_mid(probs, SCALE * smerr, mask_b, lab_b)
    smoothed, _ = _make_prop(n, c, cpw, NUM_SMOOTHING_LAYERS, 0.0, 1.0)(
        disb * y, SMOOTHING_ALPHA * disb,
        (1.0 - SMOOTHING_ALPHA) * y, disb, src3, dst3)

    return _tc_logclip(smoothed)


# speedup vs baseline: 7.4979x; 7.4979x over previous
import functools
from functools import partial

import jax
import jax.numpy as jnp
from jax import lax
from jax.experimental import pallas as pl
from jax.experimental.pallas import tpu as pltpu
from jax.experimental.pallas import tpu_sc as plsc

NW = 16
K = 128


def _cdiv(a, b):
    return (a + b - 1) // b


def _make_phases(n, c, cpw, nlayers_c, alpha_c, nlayers_s, alpha_s):
    zpw = n // (NW * 128)
    assert n == NW * 128 * zpw
    npw = n // NW
    cb = 128
    ncb = npw // cb
    mesh = plsc.VectorSubcoreMesh(core_axis_name="c", subcore_axis_name="s",
                                  num_cores=1)

    @partial(
        pl.kernel,
        out_type=(jax.ShapeDtypeStruct((n, c), jnp.float32),
                  jax.ShapeDtypeStruct((n, c), jnp.float32),
                  jax.ShapeDtypeStruct((n, c), jnp.float32)),
        mesh=mesh,
        compiler_params=pltpu.CompilerParams(use_tc_tiling_on_sc=False, needs_layout_passes=False),
        scratch_types=[
            pltpu.VMEM((cpw, K), jnp.int32),
            pltpu.VMEM((cpw, K), jnp.int32),
            pltpu.VMEM((K, c), jnp.float32),
            pltpu.VMEM((K, c), jnp.float32),
            pltpu.VMEM((128, c), jnp.float32),
            pltpu.VMEM((cb, c), jnp.float32),
            pltpu.VMEM((cb, c), jnp.float32),
            pltpu.VMEM((cb, c), jnp.float32),
            pltpu.VMEM_SHARED((n, c), jnp.float32),
            pltpu.SemaphoreType.DMA,
            pltpu.SemaphoreType.DMA,
        ],
    )
    def phases(g0, res_c, disb, probs, oh, maskf, src3, dst3,
               h_out, g_out, res2,
               sidx, didx, bufa, bufb, zbuf, accb, db, rb, acc, sema, semb):
        w = lax.axis_index("s")

        pltpu.sync_copy(src3.at[w], sidx)
        pltpu.sync_copy(dst3.at[w], didx)

        def zinit(r, carry):
            for c4 in range(c // 16):
                zbuf[r, pl.ds(c4 * 16, 16)] = jnp.zeros((16,), jnp.float32)
            return carry
        lax.fori_loop(0, 128, zinit, 0)

        for j in range(ncb):
            r0 = w * npw + j * cb
            pltpu.sync_copy(g0.at[pl.ds(r0, cb)], accb)
            pltpu.sync_copy(accb, g_out.at[pl.ds(r0, cb)])
        plsc.subcore_barrier()

        def label_prop(nlayers, alpha, res, lo, hi):
            def one_iter(i, carry):
                for z in range(zpw):
                    pltpu.sync_copy(zbuf, acc.at[pl.ds(w * npw + z * 128, 128)])
                plsc.subcore_barrier()

                def pair(p, carry2):
                    c0 = 2 * p
                    da = pltpu.async_copy(g_out.at[sidx.at[c0]], bufa, sema)
                    dbc = pltpu.async_copy(g_out.at[sidx.at[c0 + 1]], bufb,
                                           semb)
                    da.wait()
                    pltpu.sync_copy(bufa, acc.at[didx.at[c0]], add=True)
                    dbc.wait()
                    pltpu.sync_copy(bufb, acc.at[didx.at[c0 + 1]], add=True)
                    return carry2
                lax.fori_loop(0, cpw // 2, pair, 0)
                plsc.subcore_barrier()

                for j in range(ncb):
                    r0 = w * npw + j * cb
                    pltpu.sync_copy(acc.at[pl.ds(r0, cb)], accb)
                    pltpu.sync_copy(disb.at[pl.ds(r0, cb)], db)
                    pltpu.sync_copy(res.at[pl.ds(r0, cb)], rb)

                    def crow(r, carry3):
                        for c4 in range(c // 16):
                            sl = pl.ds(c4 * 16, 16)
                            dv = db[r, sl]
                            h = jnp.minimum(
                                jnp.maximum(alpha * dv * accb[r, sl]
                                            + rb[r, sl], lo), hi)
                            accb[r, sl] = h
                            db[r, sl] = dv * h
                        return carry3
                    lax.fori_loop(0, cb, crow, 0)

                    pltpu.sync_copy(accb, h_out.at[pl.ds(r0, cb)])
                    pltpu.sync_copy(db, g_out.at[pl.ds(r0, cb)])
                plsc.subcore_barrier()
                return carry
            lax.fori_loop(0, nlayers, one_iter, 0)

        label_prop(nlayers_c, alpha_c, res_c, -1.0, 1.0)

        for j in range(ncb):
            r0 = w * npw + j * cb
            pltpu.sync_copy(h_out.at[pl.ds(r0, cb)], accb)
            pltpu.sync_copy(probs.at[pl.ds(r0, cb)], rb)
            pltpu.sync_copy(oh.at[pl.ds(r0, cb)], db)
            pltpu.sync_copy(maskf.at[pl.ds(r0, cb)], bufa)
            pltpu.sync_copy(disb.at[pl.ds(r0, cb)], bufb)

            def mrow(r, carry):
                for c4 in range(c // 16):
                    sl = pl.ds(c4 * 16, 16)
                    mf = bufa[r, sl]
                    y = mf * db[r, sl] + (1.0 - mf) * (rb[r, sl]
                                                       + accb[r, sl])
                    accb[r, sl] = (1.0 - alpha_s) * y
                    db[r, sl] = bufb[r, sl] * y
                return carry
            lax.fori_loop(0, cb, mrow, 0)

            pltpu.sync_copy(accb, res2.at[pl.ds(r0, cb)])
            pltpu.sync_copy(db, g_out.at[pl.ds(r0, cb)])
        plsc.subcore_barrier()

        label_prop(nlayers_s, alpha_s, res2, 0.0, 1.0)

    return phases


def _make_first(n, c, cpw):
    zpw = n // (NW * 128)
    assert n == NW * 128 * zpw
    npw = n // NW
    cb = 128
    ncb = npw // cb
    mesh = plsc.VectorSubcoreMesh(core_axis_name="c", subcore_axis_name="s",
                                  num_cores=1)

    @partial(
        pl.kernel,
        out_type=(jax.ShapeDtypeStruct((n, c), jnp.float32),
                  jax.ShapeDtypeStruct((n, c), jnp.float32),
                  jax.ShapeDtypeStruct((n, c), jnp.float32)),
        mesh=mesh,
        compiler_params=pltpu.CompilerParams(use_tc_tiling_on_sc=False, needs_layout_passes=False),
        scratch_types=[
            pltpu.VMEM((cpw, K), jnp.int32),
            pltpu.VMEM((cpw, K), jnp.int32),
            pltpu.VMEM((K, c), jnp.float32),
            pltpu.VMEM((K, c), jnp.float32),
            pltpu.VMEM((128, c), jnp.float32),
            pltpu.VMEM((cb, c), jnp.float32),
            pltpu.VMEM((cb, c), jnp.float32),
            pltpu.VMEM_SHARED((n, c), jnp.float32),
            pltpu.SemaphoreType.DMA,
            pltpu.SemaphoreType.DMA,
        ],
    )
    def first(xw, src3, dst3, lg_out, disb_out, g_out,
              sidx, didx, bufa, bufb, zbuf, accb, tb, acc, sema, semb):
        w = lax.axis_index("s")
        pltpu.sync_copy(src3.at[w], sidx)
        pltpu.sync_copy(dst3.at[w], didx)

        def zinit(r, carry):
            for c4 in range(c // 16):
                zbuf[r, pl.ds(c4 * 16, 16)] = jnp.zeros((16,), jnp.float32)
                bufa[r, pl.ds(c4 * 16, 16)] = jnp.ones((16,), jnp.float32)
            return carry
        lax.fori_loop(0, 128, zinit, 0)

        for z in range(zpw):
            pltpu.sync_copy(zbuf, acc.at[pl.ds(w * npw + z * 128, 128)])
        plsc.subcore_barrier()

        def scat(cc, carry):
            pltpu.sync_copy(bufa, acc.at[didx.at[cc]], add=True)
            return carry
        lax.fori_loop(0, cpw, scat, 0)
        plsc.subcore_barrier()

        for j in range(ncb):
            r0 = w * npw + j * cb
            pltpu.sync_copy(acc.at[pl.ds(r0, cb)], accb)
            pltpu.sync_copy(xw.at[pl.ds(r0, cb)], tb)

            def drow(r, carry):
                for c4 in range(c // 16):
                    sl = pl.ds(c4 * 16, 16)
                    dv = accb[r, sl]
                    iy = jnp.int32(0x5F3759DF) - (
                        plsc.bitcast(dv, jnp.int32) >> 1)
                    y = plsc.bitcast(iy, jnp.float32)
                    for _ in range(3):
                        y = y * (1.5 - 0.5 * dv * y * y)
                    dis = jnp.where(dv > 0, y, 0.0)
                    accb[r, sl] = dis
                    tb[r, sl] = dis * tb[r, sl]
                return carry
            lax.fori_loop(0, cb, drow, 0)

            pltpu.sync_copy(accb, disb_out.at[pl.ds(r0, cb)])
            pltpu.sync_copy(tb, g_out.at[pl.ds(r0, cb)])
        plsc.subcore_barrier()

        for z in range(zpw):
            pltpu.sync_copy(zbuf, acc.at[pl.ds(w * npw + z * 128, 128)])
        plsc.subcore_barrier()

        def pair(p, carry2):
            c0 = 2 * p
            da = pltpu.async_copy(g_out.at[sidx.at[c0]], bufa, sema)
            dbc = pltpu.async_copy(g_out.at[sidx.at[c0 + 1]], bufb, semb)
            da.wait()
            pltpu.sync_copy(bufa, acc.at[didx.at[c0]], add=True)
            dbc.wait()
            pltpu.sync_copy(bufb, acc.at[didx.at[c0 + 1]], add=True)
            return carry2
        lax.fori_loop(0, cpw // 2, pair, 0)
        plsc.subcore_barrier()

        for j in range(ncb):
            r0 = w * npw + j * cb
            pltpu.sync_copy(acc.at[pl.ds(r0, cb)], accb)
            pltpu.sync_copy(disb_out.at[pl.ds(r0, cb)], tb)

            def lrow(r, carry):
                for c4 in range(c // 16):
                    sl = pl.ds(c4 * 16, 16)
                    accb[r, sl] = tb[r, sl] * accb[r, sl]
                return carry
            lax.fori_loop(0, cb, lrow, 0)
            pltpu.sync_copy(accb, lg_out.at[pl.ds(r0, cb)])

    return first


def _tc_matmul(x, wmat):
    m, d = x.shape
    cc = wmat.shape[1]
    bm = 1000

    def body(xr, wr, orf):
        orf[...] = jnp.dot(xr[...], wr[...],
                           preferred_element_type=jnp.float32)

    return pl.pallas_call(
        body,
        grid=(m // bm,),
        in_specs=[pl.BlockSpec((bm, d), lambda i: (i, 0)),
                  pl.BlockSpec((d, cc), lambda i: (0, 0))],
        out_specs=pl.BlockSpec((bm, cc), lambda i: (i, 0)),
        out_shape=jax.ShapeDtypeStruct((m, cc), jnp.float32),
    )(x, wmat)


def _tc_prep(logits, mask_b, lab_b):
    m, cc = logits.shape
    bm = 1000

    def body(lr, mr, br, pr, er):
        z = lr[...]
        zm = jnp.max(z, axis=1, keepdims=True)
        ez = jnp.exp(z - zm)
        p = ez / jnp.sum(ez, axis=1, keepdims=True)
        oh = (br[...] == lax.broadcasted_iota(jnp.int32, (bm, cc), 1)
              ).astype(jnp.float32)
        pr[...] = p
        er[...] = jnp.where(mr[...] != 0, oh - p, 0.0)

    return pl.pallas_call(
        body,
        grid=(m // bm,),
        in_specs=[pl.BlockSpec((bm, cc), lambda i: (i, 0))] * 3,
        out_specs=[pl.BlockSpec((bm, cc), lambda i: (i, 0))] * 2,
        out_shape=[jax.ShapeDtypeStruct((m, cc), jnp.float32)] * 2,
    )(logits, mask_b, lab_b)


def _tc_mid(probs, smerr, mask_b, lab_b):
    m, cc = probs.shape
    bm = 1000

    def body(pr, sr, mr, br, yr):
        oh = (br[...] == lax.broadcasted_iota(jnp.int32, (bm, cc), 1)
              ).astype(jnp.float32)
        yr[...] = jnp.where(mr[...] != 0, oh, pr[...] + sr[...])

    return pl.pallas_call(
        body,
        grid=(m // bm,),
        in_specs=[pl.BlockSpec((bm, cc), lambda i: (i, 0))] * 4,
        out_specs=pl.BlockSpec((bm, cc), lambda i: (i, 0)),
        out_shape=jax.ShapeDtypeStruct((m, cc), jnp.float32),
    )(probs, smerr, mask_b, lab_b)


def _tc_logclip(h):
    m, cc = h.shape
    bm = 1000

    def body(hr, orf):
        orf[...] = jnp.log(jnp.maximum(hr[...], 1e-15))

    return pl.pallas_call(
        body,
        grid=(m // bm,),
        in_specs=[pl.BlockSpec((bm, cc), lambda i: (i, 0))],
        out_specs=pl.BlockSpec((bm, cc), lambda i: (i, 0)),
        out_shape=jax.ShapeDtypeStruct((m, cc), jnp.float32),
    )(h)


NUM_CORRECTION_LAYERS = 50
CORRECTION_ALPHA = 0.5
NUM_SMOOTHING_LAYERS = 50
SMOOTHING_ALPHA = 0.8
SCALE = 1.0


def kernel(x, edge_index, W, train_mask, train_labels):
    n, d = x.shape
    c = W.shape[1]
    e = edge_index.shape[1]
    src = edge_index[0].astype(jnp.int32)
    dst = edge_index[1].astype(jnp.int32)

    cpw = _cdiv(e, NW * K)
    cpw += cpw % 2
    e_pad = NW * cpw * K
    src_p = jnp.concatenate([src, jnp.zeros((e_pad - e,), jnp.int32)])
    dst_p = jnp.concatenate([dst, jnp.full((e_pad - e,), n, jnp.int32)])
    src3 = src_p.reshape(NW, cpw, K)
    dst3 = dst_p.reshape(NW, cpw, K)

    n_pad = NW * 128 * _cdiv(n + 1, NW * 128)

    def padrows(a):
        return jnp.zeros((n_pad, c), jnp.float32).at[:n].set(a)

    xw = _tc_matmul(x, W)

    logits, disb, _ = _make_first(n_pad, c, cpw)(padrows(xw), src3, dst3)

    mask_b = jnp.broadcast_to(
        train_mask.astype(jnp.int32)[:, None], (n, c))
    lab_b = jnp.broadcast_to(
        train_labels.astype(jnp.int32)[:, None], (n, c))
    probs, err = _tc_prep(logits[:n], mask_b, lab_b)

    err_p = padrows(err)
    oh_p = padrows(
        (lab_b == lax.broadcasted_iota(jnp.int32, (n, c), 1)
         ).astype(jnp.float32))
    maskf_p = padrows(train_mask.astype(jnp.float32)[:, None]
                      * jnp.ones((n, c), jnp.float32))
    smoothed, _, _ = _make_phases(
        n_pad, c, cpw, NUM_CORRECTION_LAYERS, CORRECTION_ALPHA,
        NUM_SMOOTHING_LAYERS, SMOOTHING_ALPHA)(
        disb * err_p, (1.0 - CORRECTION_ALPHA) * err_p, disb,
        padrows(probs), oh_p, maskf_p, src3, dst3)

    return _tc_logclip(smoothed[:n])


# speedup vs baseline: 9.6087x; 1.2815x over previous
from functools import partial

import jax
import jax.numpy as jnp
from jax import lax
from jax.experimental import pallas as pl
from jax.experimental.pallas import tpu as pltpu
from jax.experimental.pallas import tpu_sc as plsc

NW = 16
NC = 2
K = 128
NB = 4


def _cdiv(a, b):
    return (a + b - 1) // b


_SC_PARAMS = pltpu.CompilerParams(use_tc_tiling_on_sc=False,
                                  needs_layout_passes=False)


def _mesh():
    return plsc.VectorSubcoreMesh(core_axis_name="c", subcore_axis_name="s",
                                  num_cores=NC)


def _edge_pass(g_out, acc, sidx, didx, bufs, sems, cpw):
    for b in range(NB):
        pltpu.async_copy(g_out.at[sidx.at[b]], bufs[b], sems[b])

    def quad(q, carry):
        c0 = NB * q
        for b in range(NB):
            cc = c0 + b
            pltpu.make_async_copy(g_out.at[sidx.at[cc]], bufs[b],
                                  sems[b]).wait()
            pltpu.sync_copy(bufs[b], acc.at[didx.at[cc]], add=True)
            nxt = cc + NB

            @pl.when(nxt < cpw)
            def _():
                pltpu.async_copy(g_out.at[sidx.at[nxt]], bufs[b], sems[b])
        return carry
    lax.fori_loop(0, cpw // NB, quad, 0)


def _make_first(n2, ch, cpw):
    n_pad = n2 // NC
    zpw = n_pad // (NW * 128)
    assert n_pad == NW * 128 * zpw
    npw = n_pad // NW
    cb = 128
    ncb = npw // cb

    @partial(
        pl.kernel,
        out_type=(jax.ShapeDtypeStruct((n2, ch), jnp.float32),
                  jax.ShapeDtypeStruct((n2, ch), jnp.float32),
                  jax.ShapeDtypeStruct((n2, ch), jnp.float32)),
        mesh=_mesh(),
        compiler_params=_SC_PARAMS,
        scratch_types=[
            pltpu.VMEM((cpw, K), jnp.int32),
            pltpu.VMEM((cpw, K), jnp.int32),
            pltpu.VMEM((NB, K, ch), jnp.float32),
            pltpu.VMEM((K, ch), jnp.float32),
            pltpu.VMEM((128, ch), jnp.float32),
            pltpu.VMEM((cb, ch), jnp.float32),
            pltpu.VMEM((cb, ch), jnp.float32),
            pltpu.VMEM_SHARED((n_pad, ch), jnp.float32),
        ] + [pltpu.SemaphoreType.DMA] * NB,
    )
    def first(xw2, src3b, dst3, lg_out, disb_out, g_out,
              sidx, didx, ring, ones, zbuf, accb, tb, acc, *sems):
        ci = lax.axis_index("c")
        w = lax.axis_index("s")
        base = ci * n_pad
        bufs = [ring.at[b] for b in range(NB)]

        pltpu.sync_copy(src3b.at[ci, w], sidx)
        pltpu.sync_copy(dst3.at[w], didx)

        def zinit(r, carry):
            for c4 in range(ch // 16):
                sl = pl.ds(c4 * 16, 16)
                zbuf[r, sl] = jnp.zeros((16,), jnp.float32)
                ones[r, sl] = jnp.ones((16,), jnp.float32)
            return carry
        lax.fori_loop(0, 128, zinit, 0)

        for z in range(zpw):
            pltpu.sync_copy(zbuf, acc.at[pl.ds(w * npw + z * 128, 128)])
        plsc.subcore_barrier()

        def scat(cc, carry):
            pltpu.sync_copy(ones, acc.at[didx.at[cc]], add=True)
            return carry
        lax.fori_loop(0, cpw, scat, 0)
        plsc.subcore_barrier()

        for j in range(ncb):
            ra = w * npw + j * cb
            rh = base + ra
            pltpu.sync_copy(acc.at[pl.ds(ra, cb)], accb)
            pltpu.sync_copy(xw2.at[pl.ds(rh, cb)], tb)

            def drow(r, carry):
                for c4 in range(ch // 16):
                    sl = pl.ds(c4 * 16, 16)
                    dv = accb[r, sl]
                    iy = jnp.int32(0x5F3759DF) - (
                        plsc.bitcast(dv, jnp.int32) >> 1)
                    y = plsc.bitcast(iy, jnp.float32)
                    for _ in range(3):
                        y = y * (1.5 - 0.5 * dv * y * y)
                    dis = jnp.where(dv > 0, y, 0.0)
                    accb[r, sl] = dis
                    tb[r, sl] = dis * tb[r, sl]
                return carry
            lax.fori_loop(0, cb, drow, 0)

            pltpu.sync_copy(accb, disb_out.at[pl.ds(rh, cb)])
            pltpu.sync_copy(tb, g_out.at[pl.ds(rh, cb)])
        plsc.subcore_barrier()

        for z in range(zpw):
            pltpu.sync_copy(zbuf, acc.at[pl.ds(w * npw + z * 128, 128)])
        plsc.subcore_barrier()
        _edge_pass(g_out, acc, sidx, didx, bufs, sems, cpw)
        plsc.subcore_barrier()

        for j in range(ncb):
            ra = w * npw + j * cb
            rh = base + ra
            pltpu.sync_copy(acc.at[pl.ds(ra, cb)], accb)
            pltpu.sync_copy(disb_out.at[pl.ds(rh, cb)], tb)

            def lrow(r, carry):
                for c4 in range(ch // 16):
                    sl = pl.ds(c4 * 16, 16)
                    accb[r, sl] = tb[r, sl] * accb[r, sl]
                return carry
            lax.fori_loop(0, cb, lrow, 0)
            pltpu.sync_copy(accb, lg_out.at[pl.ds(rh, cb)])

    return first


def _make_phases(n2, ch, cpw, nlayers_c, alpha_c, nlayers_s, alpha_s):
    n_pad = n2 // NC
    zpw = n_pad // (NW * 128)
    assert n_pad == NW * 128 * zpw
    npw = n_pad // NW
    cb = 128
    ncb = npw // cb

    @partial(
        pl.kernel,
        out_type=(jax.ShapeDtypeStruct((n2, ch), jnp.float32),
                  jax.ShapeDtypeStruct((n2, ch), jnp.float32),
                  jax.ShapeDtypeStruct((n2, ch), jnp.float32)),
        mesh=_mesh(),
        compiler_params=_SC_PARAMS,
        scratch_types=[
            pltpu.VMEM((cpw, K), jnp.int32),
            pltpu.VMEM((cpw, K), jnp.int32),
            pltpu.VMEM((NB, K, ch), jnp.float32),
            pltpu.VMEM((128, ch), jnp.float32),
            pltpu.VMEM((cb, ch), jnp.float32),
            pltpu.VMEM((cb, ch), jnp.float32),
            pltpu.VMEM((cb, ch), jnp.float32),
            pltpu.VMEM((cb, ch), jnp.float32),
            pltpu.VMEM_SHARED((n_pad, ch), jnp.float32),
        ] + [pltpu.SemaphoreType.DMA] * NB,
    )
    def phases(g0, res_c, disb, probs, oh, maskf, src3b, dst3,
               h_out, g_out, res2,
               sidx, didx, ring, zbuf, accb, db, rb, xb, acc, *sems):
        ci = lax.axis_index("c")
        w = lax.axis_index("s")
        base = ci * n_pad
        bufs = [ring.at[b] for b in range(NB)]

        pltpu.sync_copy(src3b.at[ci, w], sidx)
        pltpu.sync_copy(dst3.at[w], didx)

        def zinit(r, carry):
            for c4 in range(ch // 16):
                zbuf[r, pl.ds(c4 * 16, 16)] = jnp.zeros((16,), jnp.float32)
            return carry
        lax.fori_loop(0, 128, zinit, 0)

        for j in range(ncb):
            rh = base + w * npw + j * cb
            pltpu.sync_copy(g0.at[pl.ds(rh, cb)], accb)
            pltpu.sync_copy(accb, g_out.at[pl.ds(rh, cb)])
        plsc.subcore_barrier()

        def label_prop(nlayers, alpha, res, lo, hi):
            def one_iter(i, carry):
                for z in range(zpw):
                    pltpu.sync_copy(zbuf,
                                    acc.at[pl.ds(w * npw + z * 128, 128)])
                plsc.subcore_barrier()
                _edge_pass(g_out, acc, sidx, didx, bufs, sems, cpw)
                plsc.subcore_barrier()

                for j in range(ncb):
                    ra = w * npw + j * cb
                    rh = base + ra
                    pltpu.sync_copy(acc.at[pl.ds(ra, cb)], accb)
                    pltpu.sync_copy(disb.at[pl.ds(rh, cb)], db)
                    pltpu.sync_copy(res.at[pl.ds(rh, cb)], rb)

                    def crow(r, carry3):
                        for c4 in range(ch // 16):
                            sl = pl.ds(c4 * 16, 16)
                            dv = db[r, sl]
                            h = jnp.minimum(
                                jnp.maximum(alpha * dv * accb[r, sl]
                                            + rb[r, sl], lo), hi)
                            accb[r, sl] = h
                            db[r, sl] = dv * h
                        return carry3
                    lax.fori_loop(0, cb, crow, 0)

                    pltpu.sync_copy(accb, h_out.at[pl.ds(rh, cb)])
                    pltpu.sync_copy(db, g_out.at[pl.ds(rh, cb)])
                plsc.subcore_barrier()
                return carry
            lax.fori_loop(0, nlayers, one_iter, 0)

        label_prop(nlayers_c, alpha_c, res_c, -1.0, 1.0)

        for j in range(ncb):
            rh = base + w * npw + j * cb
            pltpu.sync_copy(h_out.at[pl.ds(rh, cb)], accb)
            pltpu.sync_copy(probs.at[pl.ds(rh, cb)], rb)
            pltpu.sync_copy(oh.at[pl.ds(rh, cb)], db)
            pltpu.sync_copy(maskf.at[pl.ds(rh, cb)], xb)
            pltpu.sync_copy(disb.at[pl.ds(rh, cb)], zbuf)

            def mrow(r, carry):
                for c4 in range(ch // 16):
                    sl = pl.ds(c4 * 16, 16)
                    mf = xb[r, sl]
                    y = mf * db[r, sl] + (1.0 - mf) * (rb[r, sl]
                                                       + accb[r, sl])
                    accb[r, sl] = (1.0 - alpha_s) * y
                    db[r, sl] = zbuf[r, sl] * y
                return carry
            lax.fori_loop(0, cb, mrow, 0)

            pltpu.sync_copy(accb, res2.at[pl.ds(rh, cb)])
            pltpu.sync_copy(db, g_out.at[pl.ds(rh, cb)])

        def zinit2(r, carry):
            for c4 in range(ch // 16):
                zbuf[r, pl.ds(c4 * 16, 16)] = jnp.zeros((16,), jnp.float32)
            return carry
        lax.fori_loop(0, 128, zinit2, 0)
        plsc.subcore_barrier()

        label_prop(nlayers_s, alpha_s, res2, 0.0, 1.0)

    return phases


def _tc_matmul(x, wmat):
    m, d = x.shape
    cc = wmat.shape[1]
    bm = 1000

    def body(xr, wr, orf):
        orf[...] = jnp.dot(xr[...], wr[...],
                           preferred_element_type=jnp.float32)

    return pl.pallas_call(
        body,
        grid=(m // bm,),
        in_specs=[pl.BlockSpec((bm, d), lambda i: (i, 0)),
                  pl.BlockSpec((d, cc), lambda i: (0, 0))],
        out_specs=pl.BlockSpec((bm, cc), lambda i: (i, 0)),
        out_shape=jax.ShapeDtypeStruct((m, cc), jnp.float32),
    )(x, wmat)


def _tc_prep(logits, mask_b, lab_b):
    m, cc = logits.shape
    bm = 1000

    def body(lr, mr, br, pr, er):
        z = lr[...]
        zm = jnp.max(z, axis=1, keepdims=True)
        ez = jnp.exp(z - zm)
        p = ez / jnp.sum(ez, axis=1, keepdims=True)
        oh = (br[...] == lax.broadcasted_iota(jnp.int32, (bm, cc), 1)
              ).astype(jnp.float32)
        pr[...] = p
        er[...] = jnp.where(mr[...] != 0, oh - p, 0.0)

    return pl.pallas_call(
        body,
        grid=(m // bm,),
        in_specs=[pl.BlockSpec((bm, cc), lambda i: (i, 0))] * 3,
        out_specs=[pl.BlockSpec((bm, cc), lambda i: (i, 0))] * 2,
        out_shape=[jax.ShapeDtypeStruct((m, cc), jnp.float32)] * 2,
    )(logits, mask_b, lab_b)


def _tc_logclip(h):
    m, cc = h.shape
    bm = 1000

    def body(hr, orf):
        orf[...] = jnp.log(jnp.maximum(hr[...], 1e-15))

    return pl.pallas_call(
        body,
        grid=(m // bm,),
        in_specs=[pl.BlockSpec((bm, cc), lambda i: (i, 0))],
        out_specs=pl.BlockSpec((bm, cc), lambda i: (i, 0)),
        out_shape=jax.ShapeDtypeStruct((m, cc), jnp.float32),
    )(h)


NUM_CORRECTION_LAYERS = 50
CORRECTION_ALPHA = 0.5
NUM_SMOOTHING_LAYERS = 50
SMOOTHING_ALPHA = 0.8
SCALE = 1.0


def kernel(x, edge_index, W, train_mask, train_labels):
    n, d = x.shape
    c = W.shape[1]
    e = edge_index.shape[1]
    ch = c // NC
    src = edge_index[0].astype(jnp.int32)
    dst = edge_index[1].astype(jnp.int32)

    cpw = _cdiv(e, NW * K)
    cpw = _cdiv(cpw, NB) * NB
    e_pad = NW * cpw * K
    src_p = jnp.concatenate([src, jnp.zeros((e_pad - e,), jnp.int32)])
    dst_p = jnp.concatenate([dst, jnp.full((e_pad - e,), n, jnp.int32)])
    src3 = src_p.reshape(NW, cpw, K)
    dst3 = dst_p.reshape(NW, cpw, K)

    n_pad = NW * 128 * _cdiv(n + 1, NW * 128)
    n2 = NC * n_pad
    src3b = jnp.stack([src3, src3 + n_pad])

    def stack2(a):
        out = jnp.zeros((n2, ch), jnp.float32)
        for i in range(NC):
            out = out.at[i * n_pad:i * n_pad + n].set(
                a[:, i * ch:(i + 1) * ch])
        return out

    def unstack(s2):
        return jnp.concatenate(
            [s2[i * n_pad:i * n_pad + n] for i in range(NC)], axis=1)

    xw = _tc_matmul(x, W)

    lg2, disb2, _ = _make_first(n2, ch, cpw)(stack2(xw), src3b, dst3)
    logits = unstack(lg2)

    mask_b = jnp.broadcast_to(
        train_mask.astype(jnp.int32)[:, None], (n, c))
    lab_b = jnp.broadcast_to(
        train_labels.astype(jnp.int32)[:, None], (n, c))
    probs, err = _tc_prep(logits, mask_b, lab_b)

    err2 = stack2(err)
    oh2 = stack2((lab_b == lax.broadcasted_iota(jnp.int32, (n, c), 1)
                  ).astype(jnp.float32))
    maskf2 = stack2(jnp.broadcast_to(
        train_mask.astype(jnp.float32)[:, None], (n, c)))
    h2, _, _ = _make_phases(
        n2, ch, cpw, NUM_CORRECTION_LAYERS, CORRECTION_ALPHA,
        NUM_SMOOTHING_LAYERS, SMOOTHING_ALPHA)(
        disb2 * err2, (1.0 - CORRECTION_ALPHA) * err2, disb2,
        stack2(probs), oh2, maskf2, src3b, dst3)

    return _tc_logclip(unstack(h2))
